# baseline (device time: 235197 ns/iter reference)
import jax
import jax.numpy as jnp
from jax import lax
from jax.experimental import pallas as pl
from jax.experimental.pallas import tpu as pltpu

N_Z = 4


def kernel(ids, E):
    T = ids.shape[0]
    V_loc, D = E.shape

    z = lax.axis_index("z")
    local = ids - z * V_loc
    mask = (local >= 0) & (local < V_loc)
    safe = jnp.where(mask, local, 0)
    partial = jnp.where(mask[:, None], E[safe, :], 0.0).astype(jnp.bfloat16)

    def body(p_ref, out_ref, comm_ref, send_sems, recv_sems):
        my_x = lax.axis_index("x")
        my_y = lax.axis_index("y")
        my_z = lax.axis_index("z")
        right = lax.rem(my_z + 1, N_Z)
        left = lax.rem(my_z + N_Z - 1, N_Z)

        barrier_sem = pltpu.get_barrier_semaphore()
        for nbr in (left, right):
            pl.semaphore_signal(
                barrier_sem,
                inc=1,
                device_id=(my_x, my_y, nbr),
                device_id_type=pl.DeviceIdType.MESH,
            )
        pl.semaphore_wait(barrier_sem, 2)

        comm_ref[0, :, :] = p_ref[:, :]
        out_ref[:, :] = p_ref[:, :].astype(jnp.float32)

        for h in range(N_Z - 1):
            rdma = pltpu.make_async_remote_copy(
                src_ref=comm_ref.at[h],
                dst_ref=comm_ref.at[h + 1],
                send_sem=send_sems.at[h],
                recv_sem=recv_sems.at[h + 1],
                device_id=(my_x, my_y, right),
                device_id_type=pl.DeviceIdType.MESH,
            )
            rdma.start()
            rdma.wait()
            out_ref[:, :] = out_ref[:, :] + comm_ref[h + 1, :, :].astype(
                jnp.float32
            )

    return pl.pallas_call(
        body,
        out_shape=jax.ShapeDtypeStruct((T, D), jnp.float32),
        in_specs=[pl.BlockSpec(memory_space=pltpu.VMEM)],
        out_specs=pl.BlockSpec(memory_space=pltpu.VMEM),
        scratch_shapes=[
            pltpu.VMEM((N_Z, T, D), jnp.bfloat16),
            pltpu.SemaphoreType.DMA((N_Z,)),
            pltpu.SemaphoreType.DMA((N_Z,)),
        ],
        compiler_params=pltpu.CompilerParams(collective_id=0),
    )(partial)


# device time: 91746 ns/iter; 2.5636x vs baseline; 2.5636x over previous
import jax
import jax.numpy as jnp
from jax import lax
from jax.experimental import pallas as pl
from jax.experimental.pallas import tpu as pltpu

N_Z = 4
MESH = pl.DeviceIdType.MESH


def kernel(ids, E):
    T = ids.shape[0]
    V_loc, D = E.shape
    Tq = T // 4

    x = lax.axis_index("x")
    y = lax.axis_index("y")
    z = lax.axis_index("z")
    q = 2 * x + y

    ids_q = lax.dynamic_slice(ids, (q * Tq,), (Tq,))
    local = ids_q - z * V_loc
    mask = (local >= 0) & (local < V_loc)
    safe = jnp.where(mask, local, 0)
    partial = jnp.where(
        mask[:, None], E[safe, :].astype(jnp.bfloat16), jnp.bfloat16(0)
    )

    def body(p_ref, out_ref, zbuf, qbuf, zsend, zrecv, ysend, yrecv, xsend, xrecv):
        my_x = lax.axis_index("x")
        my_y = lax.axis_index("y")
        my_z = lax.axis_index("z")

        zbuf[1:, :, :] = jnp.zeros((2 * (N_Z - 1), Tq, D), jnp.bfloat16)
        zbuf[0, :, :] = p_ref[:, :]

        barrier_sem = pltpu.get_barrier_semaphore()
        for dev in (
            (my_x, my_y, lax.rem(my_z + 1, N_Z)),
            (my_x, my_y, lax.rem(my_z + N_Z - 1, N_Z)),
            (my_x, 1 - my_y, my_z),
            (1 - my_x, my_y, my_z),
        ):
            pl.semaphore_signal(
                barrier_sem, inc=1, device_id=dev, device_id_type=MESH
            )
        pl.semaphore_wait(barrier_sem, 4)

        for s in range(1, N_Z):
            src_r = 0 if s == 1 else s - 1
            dst_r = s
            src_l = 0 if s == 1 else N_Z - 1 + s - 1
            dst_l = N_Z - 1 + s

            send_right = (my_z < N_Z - 1) & (my_z >= s - 1)
            send_left = (my_z > 0) & (my_z <= N_Z - s)
            recv_left = my_z >= s
            recv_right = my_z <= N_Z - 1 - s

            @pl.when(send_right)
            def _(src_r=src_r, dst_r=dst_r, s=s):
                pltpu.make_async_remote_copy(
                    src_ref=zbuf.at[src_r],
                    dst_ref=zbuf.at[dst_r],
                    send_sem=zsend.at[s - 1, 0],
                    recv_sem=zrecv.at[s - 1, 0],
                    device_id=(my_x, my_y, my_z + 1),
                    device_id_type=MESH,
                ).start()

            @pl.when(send_left)
            def _(src_l=src_l, dst_l=dst_l, s=s):
                pltpu.make_async_remote_copy(
                    src_ref=zbuf.at[src_l],
                    dst_ref=zbuf.at[dst_l],
                    send_sem=zsend.at[s - 1, 1],
                    recv_sem=zrecv.at[s - 1, 1],
                    device_id=(my_x, my_y, my_z - 1),
                    device_id_type=MESH,
                ).start()

            @pl.when(recv_left)
            def _(dst_r=dst_r, s=s):
                pltpu.make_async_remote_copy(
                    src_ref=zbuf.at[0],
                    dst_ref=zbuf.at[dst_r],
                    send_sem=zsend.at[s - 1, 0],
                    recv_sem=zrecv.at[s - 1, 0],
                    device_id=(my_x, my_y, my_z),
                    device_id_type=MESH,
                ).wait_recv()

            @pl.when(recv_right)
            def _(dst_l=dst_l, s=s):
                pltpu.make_async_remote_copy(
                    src_ref=zbuf.at[0],
                    dst_ref=zbuf.at[dst_l],
                    send_sem=zsend.at[s - 1, 1],
                    recv_sem=zrecv.at[s - 1, 1],
                    device_id=(my_x, my_y, my_z),
                    device_id_type=MESH,
                ).wait_recv()

            @pl.when(send_right)
            def _(src_r=src_r, s=s):
                pltpu.make_async_remote_copy(
                    src_ref=zbuf.at[src_r],
                    dst_ref=zbuf.at[0],
                    send_sem=zsend.at[s - 1, 0],
                    recv_sem=zrecv.at[s - 1, 0],
                    device_id=(my_x, my_y, my_z),
                    device_id_type=MESH,
                ).wait_send()

            @pl.when(send_left)
            def _(src_l=src_l, s=s):
                pltpu.make_async_remote_copy(
                    src_ref=zbuf.at[src_l],
                    dst_ref=zbuf.at[0],
                    send_sem=zsend.at[s - 1, 1],
                    recv_sem=zrecv.at[s - 1, 1],
                    device_id=(my_x, my_y, my_z),
                    device_id_type=MESH,
                ).wait_send()

        acc = zbuf[0, :, :]
        for i in range(1, 2 * N_Z - 1):
            acc = acc + zbuf[i, :, :]

        qbuf[0, :, :] = acc

        y_peer = (my_x, 1 - my_y, my_z)
        x_peer = (1 - my_x, my_y, my_z)

        pltpu.make_async_remote_copy(
            src_ref=qbuf.at[0], dst_ref=qbuf.at[1],
            send_sem=ysend.at[0], recv_sem=yrecv.at[0],
            device_id=y_peer, device_id_type=MESH,
        ).start()
        pltpu.make_async_remote_copy(
            src_ref=qbuf.at[0], dst_ref=qbuf.at[2],
            send_sem=xsend.at[0], recv_sem=xrecv.at[0],
            device_id=x_peer, device_id_type=MESH,
        ).start()

        pltpu.make_async_remote_copy(
            src_ref=qbuf.at[0], dst_ref=qbuf.at[1],
            send_sem=ysend.at[0], recv_sem=yrecv.at[0],
            device_id=y_peer, device_id_type=MESH,
        ).wait_recv()

        pltpu.make_async_remote_copy(
            src_ref=qbuf.at[1], dst_ref=qbuf.at[3],
            send_sem=xsend.at[1], recv_sem=xrecv.at[1],
            device_id=x_peer, device_id_type=MESH,
        ).start()

        for slot, sem in ((2, 0), (3, 1)):
            pltpu.make_async_remote_copy(
                src_ref=qbuf.at[0], dst_ref=qbuf.at[slot],
                send_sem=xsend.at[sem], recv_sem=xrecv.at[sem],
                device_id=x_peer, device_id_type=MESH,
            ).wait_recv()

        for src, sems, sem_i in ((0, ysend, 0), (0, xsend, 0), (1, xsend, 1)):
            pltpu.make_async_remote_copy(
                src_ref=qbuf.at[src], dst_ref=qbuf.at[0],
                send_sem=sems.at[sem_i], recv_sem=yrecv.at[0],
                device_id=(my_x, my_y, my_z), device_id_type=MESH,
            ).wait_send()

        quarters = (
            2 * my_x + my_y,
            2 * my_x + 1 - my_y,
            2 * (1 - my_x) + my_y,
            2 * (1 - my_x) + 1 - my_y,
        )
        for k, qk in enumerate(quarters):
            out_ref[pl.ds(qk * Tq, Tq), :] = qbuf[k, :, :].astype(jnp.float32)

    return pl.pallas_call(
        body,
        out_shape=jax.ShapeDtypeStruct((T, D), jnp.float32),
        in_specs=[pl.BlockSpec(memory_space=pltpu.VMEM)],
        out_specs=pl.BlockSpec(memory_space=pltpu.VMEM),
        scratch_shapes=[
            pltpu.VMEM((2 * N_Z - 1, Tq, D), jnp.bfloat16),
            pltpu.VMEM((4, Tq, D), jnp.bfloat16),
            pltpu.SemaphoreType.DMA((N_Z - 1, 2)),
            pltpu.SemaphoreType.DMA((N_Z - 1, 2)),
            pltpu.SemaphoreType.DMA((1,)),
            pltpu.SemaphoreType.DMA((1,)),
            pltpu.SemaphoreType.DMA((2,)),
            pltpu.SemaphoreType.DMA((2,)),
        ],
        compiler_params=pltpu.CompilerParams(collective_id=0),
    )(partial)


# device time: 76786 ns/iter; 3.0630x vs baseline; 1.1948x over previous
import jax
import jax.numpy as jnp
from jax import lax
from jax.experimental import pallas as pl
from jax.experimental.pallas import tpu as pltpu

N_Z = 4
NSLOT = 2 * N_Z - 1
MESH = pl.DeviceIdType.MESH


def kernel(ids, E):
    T = ids.shape[0]
    V_loc, D = E.shape
    Tq = T // 4
    Th = Tq // 2

    x = lax.axis_index("x")
    y = lax.axis_index("y")
    z = lax.axis_index("z")
    q = 2 * x + y

    ids_q = lax.dynamic_slice(ids, (q * Tq,), (Tq,))
    local = ids_q - z * V_loc
    mask = (local >= 0) & (local < V_loc)
    safe = jnp.where(mask, local, 0)
    partial = jnp.where(
        mask[:, None], E[safe, :].astype(jnp.bfloat16), jnp.bfloat16(0)
    )

    def body(p_ref, out_ref, zbuf, qbuf, zsend, zrecv, ysend, yrecv, xsend, xrecv):
        my_x = lax.axis_index("x")
        my_y = lax.axis_index("y")
        my_z = lax.axis_index("z")
        y_peer = (my_x, 1 - my_y, my_z)
        x_peer = (1 - my_x, my_y, my_z)

        for h in range(2):
            zbuf[h, 1:, :, :] = jnp.zeros((NSLOT - 1, Th, D), jnp.bfloat16)
            zbuf[h, 0, :, :] = p_ref[h * Th : (h + 1) * Th, :]

        barrier_sem = pltpu.get_barrier_semaphore()
        for dev in (
            (my_x, my_y, lax.rem(my_z + 1, N_Z)),
            (my_x, my_y, lax.rem(my_z + N_Z - 1, N_Z)),
            y_peer,
            x_peer,
        ):
            pl.semaphore_signal(
                barrier_sem, inc=1, device_id=dev, device_id_type=MESH
            )
        pl.semaphore_wait(barrier_sem, 4)

        def z_issue(h, s):
            src_r = 0 if s == 1 else s - 1
            src_l = 0 if s == 1 else N_Z - 1 + s - 1

            @pl.when((my_z < N_Z - 1) & (my_z >= s - 1))
            def _():
                pltpu.make_async_remote_copy(
                    src_ref=zbuf.at[h, src_r],
                    dst_ref=zbuf.at[h, s],
                    send_sem=zsend.at[h, s - 1, 0],
                    recv_sem=zrecv.at[h, s - 1, 0],
                    device_id=(my_x, my_y, my_z + 1),
                    device_id_type=MESH,
                ).start()

            @pl.when((my_z > 0) & (my_z <= N_Z - s))
            def _():
                pltpu.make_async_remote_copy(
                    src_ref=zbuf.at[h, src_l],
                    dst_ref=zbuf.at[h, N_Z - 1 + s],
                    send_sem=zsend.at[h, s - 1, 1],
                    recv_sem=zrecv.at[h, s - 1, 1],
                    device_id=(my_x, my_y, my_z - 1),
                    device_id_type=MESH,
                ).start()

        def z_wait(h, s):
            @pl.when(my_z >= s)
            def _():
                pltpu.make_async_remote_copy(
                    src_ref=zbuf.at[h, 0],
                    dst_ref=zbuf.at[h, s],
                    send_sem=zsend.at[h, s - 1, 0],
                    recv_sem=zrecv.at[h, s - 1, 0],
                    device_id=(my_x, my_y, my_z),
                    device_id_type=MESH,
                ).wait_recv()

            @pl.when(my_z <= N_Z - 1 - s)
            def _():
                pltpu.make_async_remote_copy(
                    src_ref=zbuf.at[h, 0],
                    dst_ref=zbuf.at[h, N_Z - 1 + s],
                    send_sem=zsend.at[h, s - 1, 1],
                    recv_sem=zrecv.at[h, s - 1, 1],
                    device_id=(my_x, my_y, my_z),
                    device_id_type=MESH,
                ).wait_recv()

            src_r = 0 if s == 1 else s - 1
            src_l = 0 if s == 1 else N_Z - 1 + s - 1

            @pl.when((my_z < N_Z - 1) & (my_z >= s - 1))
            def _():
                pltpu.make_async_remote_copy(
                    src_ref=zbuf.at[h, src_r],
                    dst_ref=zbuf.at[h, 0],
                    send_sem=zsend.at[h, s - 1, 0],
                    recv_sem=zrecv.at[h, s - 1, 0],
                    device_id=(my_x, my_y, my_z),
                    device_id_type=MESH,
                ).wait_send()

            @pl.when((my_z > 0) & (my_z <= N_Z - s))
            def _():
                pltpu.make_async_remote_copy(
                    src_ref=zbuf.at[h, src_l],
                    dst_ref=zbuf.at[h, 0],
                    send_sem=zsend.at[h, s - 1, 1],
                    recv_sem=zrecv.at[h, s - 1, 1],
                    device_id=(my_x, my_y, my_z),
                    device_id_type=MESH,
                ).wait_send()

        def qcopy(h, src_slot, dst_slot, sems_s, sems_r, si, peer):
            return pltpu.make_async_remote_copy(
                src_ref=qbuf.at[h, src_slot],
                dst_ref=qbuf.at[h, dst_slot],
                send_sem=sems_s.at[h, si],
                recv_sem=sems_r.at[h, si],
                device_id=peer,
                device_id_type=MESH,
            )

        def half_sum(h):
            acc = zbuf[h, 0, :, :]
            for i in range(1, NSLOT):
                acc = acc + zbuf[h, i, :, :]
            qbuf[h, 0, :, :] = acc

        z_issue(0, 1)
        z_issue(1, 1)
        z_wait(0, 1)
        z_issue(0, 2)
        z_wait(1, 1)
        z_issue(1, 2)
        z_wait(0, 2)
        z_issue(0, 3)
        z_wait(1, 2)
        z_issue(1, 3)
        z_wait(0, 3)

        half_sum(0)
        qcopy(0, 0, 1, ysend, yrecv, 0, y_peer).start()
        qcopy(0, 0, 2, xsend, xrecv, 0, x_peer).start()

        z_wait(1, 3)

        half_sum(1)
        qcopy(1, 0, 1, ysend, yrecv, 0, y_peer).start()
        qcopy(1, 0, 2, xsend, xrecv, 0, x_peer).start()

        qcopy(0, 0, 1, ysend, yrecv, 0, y_peer).wait_recv()
        qcopy(0, 1, 3, xsend, xrecv, 1, x_peer).start()
        qcopy(1, 0, 2, xsend, xrecv, 0, x_peer).wait_recv()
        qcopy(1, 2, 3, ysend, yrecv, 1, y_peer).start()

        qcopy(0, 0, 2, xsend, xrecv, 0, x_peer).wait_recv()
        qcopy(0, 1, 3, xsend, xrecv, 1, x_peer).wait_recv()
        qcopy(1, 0, 1, ysend, yrecv, 0, y_peer).wait_recv()
        qcopy(1, 2, 3, ysend, yrecv, 1, y_peer).wait_recv()

        qcopy(0, 0, 0, ysend, yrecv, 0, y_peer).wait_send()
        qcopy(0, 0, 0, xsend, xrecv, 0, x_peer).wait_send()
        qcopy(0, 1, 0, xsend, xrecv, 1, x_peer).wait_send()
        qcopy(1, 0, 0, ysend, yrecv, 0, y_peer).wait_send()
        qcopy(1, 0, 0, xsend, xrecv, 0, x_peer).wait_send()
        qcopy(1, 2, 0, ysend, yrecv, 1, y_peer).wait_send()

        quarters = (
            2 * my_x + my_y,
            2 * my_x + 1 - my_y,
            2 * (1 - my_x) + my_y,
            2 * (1 - my_x) + 1 - my_y,
        )
        for h in range(2):
            for k, qk in enumerate(quarters):
                out_ref[pl.ds(qk * Tq + h * Th, Th), :] = qbuf[h, k, :, :].astype(
                    jnp.float32
                )

    return pl.pallas_call(
        body,
        out_shape=jax.ShapeDtypeStruct((T, D), jnp.float32),
        in_specs=[pl.BlockSpec(memory_space=pltpu.VMEM)],
        out_specs=pl.BlockSpec(memory_space=pltpu.VMEM),
        scratch_shapes=[
            pltpu.VMEM((2, NSLOT, Th, D), jnp.bfloat16),
            pltpu.VMEM((2, 4, Th, D), jnp.bfloat16),
            pltpu.SemaphoreType.DMA((2, N_Z - 1, 2)),
            pltpu.SemaphoreType.DMA((2, N_Z - 1, 2)),
            pltpu.SemaphoreType.DMA((2, 2)),
            pltpu.SemaphoreType.DMA((2, 2)),
            pltpu.SemaphoreType.DMA((2, 2)),
            pltpu.SemaphoreType.DMA((2, 2)),
        ],
        compiler_params=pltpu.CompilerParams(collective_id=0),
    )(partial)


# device time: 70812 ns/iter; 3.3214x vs baseline; 1.0844x over previous
import jax
import jax.numpy as jnp
from jax import lax
from jax.experimental import pallas as pl
from jax.experimental.pallas import tpu as pltpu

N_Z = 4
NSLOT = 2 * N_Z - 1
MESH = pl.DeviceIdType.MESH


def kernel(ids, E):
    T = ids.shape[0]
    V_loc, D = E.shape
    Tq = T // 4
    Th = Tq // 2

    x = lax.axis_index("x")
    y = lax.axis_index("y")
    z = lax.axis_index("z")
    q = 2 * x + y

    ids_q = lax.dynamic_slice(ids, (q * Tq,), (Tq,))
    local = ids_q - z * V_loc
    mask = (local >= 0) & (local < V_loc)
    safe = jnp.where(mask, local, 0).astype(jnp.int32)
    maskf = mask.astype(jnp.float32)[:, None]

    def body(
        safe_ref,
        E_ref,
        mask_ref,
        out_ref,
        gbuf,
        gsem,
        zbuf,
        qbuf,
        zsend,
        zrecv,
        ysend,
        yrecv,
        xsend,
        xrecv,
    ):
        my_x = lax.axis_index("x")
        my_y = lax.axis_index("y")
        my_z = lax.axis_index("z")
        y_peer = (my_x, 1 - my_y, my_z)
        x_peer = (1 - my_x, my_y, my_z)

        def gather_issue(h):
            def lp(t, c):
                idx = safe_ref[h * Th + t]
                pltpu.make_async_copy(
                    E_ref.at[idx], gbuf.at[h, t], gsem.at[h]
                ).start()
                return c

            lax.fori_loop(0, Th, lp, 0, unroll=8)

        def gather_finish(h):
            pltpu.make_async_copy(
                E_ref.at[pl.ds(0, Th)], gbuf.at[h], gsem.at[h]
            ).wait()
            zbuf[h, 0, :, :] = (
                gbuf[h, :, :] * mask_ref[h * Th : (h + 1) * Th, :]
            ).astype(jnp.bfloat16)

        for h in range(2):
            zbuf[h, 1:, :, :] = jnp.zeros((NSLOT - 1, Th, D), jnp.bfloat16)

        gather_issue(0)

        barrier_sem = pltpu.get_barrier_semaphore()
        for dev in (
            (my_x, my_y, lax.rem(my_z + 1, N_Z)),
            (my_x, my_y, lax.rem(my_z + N_Z - 1, N_Z)),
            y_peer,
            x_peer,
        ):
            pl.semaphore_signal(
                barrier_sem, inc=1, device_id=dev, device_id_type=MESH
            )
        pl.semaphore_wait(barrier_sem, 4)

        def z_issue(h, s):
            src_r = 0 if s == 1 else s - 1
            src_l = 0 if s == 1 else N_Z - 1 + s - 1

            @pl.when((my_z < N_Z - 1) & (my_z >= s - 1))
            def _():
                pltpu.make_async_remote_copy(
                    src_ref=zbuf.at[h, src_r],
                    dst_ref=zbuf.at[h, s],
                    send_sem=zsend.at[h, s - 1, 0],
                    recv_sem=zrecv.at[h, s - 1, 0],
                    device_id=(my_x, my_y, my_z + 1),
                    device_id_type=MESH,
                ).start()

            @pl.when((my_z > 0) & (my_z <= N_Z - s))
            def _():
                pltpu.make_async_remote_copy(
                    src_ref=zbuf.at[h, src_l],
                    dst_ref=zbuf.at[h, N_Z - 1 + s],
                    send_sem=zsend.at[h, s - 1, 1],
                    recv_sem=zrecv.at[h, s - 1, 1],
                    device_id=(my_x, my_y, my_z - 1),
                    device_id_type=MESH,
                ).start()

        def z_wait(h, s):
            @pl.when(my_z >= s)
            def _():
                pltpu.make_async_remote_copy(
                    src_ref=zbuf.at[h, 0],
                    dst_ref=zbuf.at[h, s],
                    send_sem=zsend.at[h, s - 1, 0],
                    recv_sem=zrecv.at[h, s - 1, 0],
                    device_id=(my_x, my_y, my_z),
                    device_id_type=MESH,
                ).wait_recv()

            @pl.when(my_z <= N_Z - 1 - s)
            def _():
                pltpu.make_async_remote_copy(
                    src_ref=zbuf.at[h, 0],
                    dst_ref=zbuf.at[h, N_Z - 1 + s],
                    send_sem=zsend.at[h, s - 1, 1],
                    recv_sem=zrecv.at[h, s - 1, 1],
                    device_id=(my_x, my_y, my_z),
                    device_id_type=MESH,
                ).wait_recv()

            src_r = 0 if s == 1 else s - 1
            src_l = 0 if s == 1 else N_Z - 1 + s - 1

            @pl.when((my_z < N_Z - 1) & (my_z >= s - 1))
            def _():
                pltpu.make_async_remote_copy(
                    src_ref=zbuf.at[h, src_r],
                    dst_ref=zbuf.at[h, 0],
                    send_sem=zsend.at[h, s - 1, 0],
                    recv_sem=zrecv.at[h, s - 1, 0],
                    device_id=(my_x, my_y, my_z),
                    device_id_type=MESH,
                ).wait_send()

            @pl.when((my_z > 0) & (my_z <= N_Z - s))
            def _():
                pltpu.make_async_remote_copy(
                    src_ref=zbuf.at[h, src_l],
                    dst_ref=zbuf.at[h, 0],
                    send_sem=zsend.at[h, s - 1, 1],
                    recv_sem=zrecv.at[h, s - 1, 1],
                    device_id=(my_x, my_y, my_z),
                    device_id_type=MESH,
                ).wait_send()

        def qcopy(h, src_slot, dst_slot, sems_s, sems_r, si, peer):
            return pltpu.make_async_remote_copy(
                src_ref=qbuf.at[h, src_slot],
                dst_ref=qbuf.at[h, dst_slot],
                send_sem=sems_s.at[h, si],
                recv_sem=sems_r.at[h, si],
                device_id=peer,
                device_id_type=MESH,
            )

        def half_sum(h):
            acc = zbuf[h, 0, :, :]
            for i in range(1, NSLOT):
                acc = acc + zbuf[h, i, :, :]
            qbuf[h, 0, :, :] = acc

        gather_finish(0)
        z_issue(0, 1)
        gather_issue(1)
        gather_finish(1)
        z_issue(1, 1)
        z_wait(0, 1)
        z_issue(0, 2)
        z_wait(1, 1)
        z_issue(1, 2)
        z_wait(0, 2)
        z_issue(0, 3)
        z_wait(1, 2)
        z_issue(1, 3)
        z_wait(0, 3)

        half_sum(0)
        qcopy(0, 0, 1, ysend, yrecv, 0, y_peer).start()
        qcopy(0, 0, 2, xsend, xrecv, 0, x_peer).start()

        z_wait(1, 3)

        half_sum(1)
        qcopy(1, 0, 1, ysend, yrecv, 0, y_peer).start()
        qcopy(1, 0, 2, xsend, xrecv, 0, x_peer).start()

        qcopy(0, 0, 1, ysend, yrecv, 0, y_peer).wait_recv()
        qcopy(0, 1, 3, xsend, xrecv, 1, x_peer).start()
        qcopy(1, 0, 2, xsend, xrecv, 0, x_peer).wait_recv()
        qcopy(1, 2, 3, ysend, yrecv, 1, y_peer).start()

        qcopy(0, 0, 2, xsend, xrecv, 0, x_peer).wait_recv()
        qcopy(0, 1, 3, xsend, xrecv, 1, x_peer).wait_recv()
        qcopy(1, 0, 1, ysend, yrecv, 0, y_peer).wait_recv()
        qcopy(1, 2, 3, ysend, yrecv, 1, y_peer).wait_recv()

        qcopy(0, 0, 0, ysend, yrecv, 0, y_peer).wait_send()
        qcopy(0, 0, 0, xsend, xrecv, 0, x_peer).wait_send()
        qcopy(0, 1, 0, xsend, xrecv, 1, x_peer).wait_send()
        qcopy(1, 0, 0, ysend, yrecv, 0, y_peer).wait_send()
        qcopy(1, 0, 0, xsend, xrecv, 0, x_peer).wait_send()
        qcopy(1, 2, 0, ysend, yrecv, 1, y_peer).wait_send()

        quarters = (
            2 * my_x + my_y,
            2 * my_x + 1 - my_y,
            2 * (1 - my_x) + my_y,
            2 * (1 - my_x) + 1 - my_y,
        )
        for h in range(2):
            for k, qk in enumerate(quarters):
                out_ref[pl.ds(qk * Tq + h * Th, Th), :] = qbuf[h, k, :, :].astype(
                    jnp.float32
                )

    grid_spec = pltpu.PrefetchScalarGridSpec(
        num_scalar_prefetch=1,
        in_specs=[
            pl.BlockSpec(memory_space=pl.ANY),
            pl.BlockSpec(memory_space=pltpu.VMEM),
        ],
        out_specs=pl.BlockSpec(memory_space=pltpu.VMEM),
        scratch_shapes=[
            pltpu.VMEM((2, Th, D), jnp.float32),
            pltpu.SemaphoreType.DMA((2,)),
            pltpu.VMEM((2, NSLOT, Th, D), jnp.bfloat16),
            pltpu.VMEM((2, 4, Th, D), jnp.bfloat16),
            pltpu.SemaphoreType.DMA((2, N_Z - 1, 2)),
            pltpu.SemaphoreType.DMA((2, N_Z - 1, 2)),
            pltpu.SemaphoreType.DMA((2, 2)),
            pltpu.SemaphoreType.DMA((2, 2)),
            pltpu.SemaphoreType.DMA((2, 2)),
            pltpu.SemaphoreType.DMA((2, 2)),
        ],
    )

    return pl.pallas_call(
        body,
        out_shape=jax.ShapeDtypeStruct((T, D), jnp.float32),
        grid_spec=grid_spec,
        compiler_params=pltpu.CompilerParams(collective_id=0),
    )(safe, E, maskf)


# device time: 69999 ns/iter; 3.3600x vs baseline; 1.0116x over previous
import jax
import jax.numpy as jnp
from jax import lax
from jax.experimental import pallas as pl
from jax.experimental.pallas import tpu as pltpu

N_Z = 4
NSLOT = 2 * N_Z - 1
MESH = pl.DeviceIdType.MESH


def kernel(ids, E):
    T = ids.shape[0]
    V_loc, D = E.shape
    Tq = T // 4
    Th = Tq // 2

    x = lax.axis_index("x")
    y = lax.axis_index("y")
    z = lax.axis_index("z")
    q = 2 * x + y

    ids_q = lax.dynamic_slice(ids, (q * Tq,), (Tq,))
    local = ids_q - z * V_loc
    mask = (local >= 0) & (local < V_loc)
    safe = jnp.where(mask, local, 0).astype(jnp.int32)
    maskf = mask.astype(jnp.float32)[:, None]

    def body(
        safe_ref,
        E_ref,
        mask_ref,
        out_ref,
        gbuf,
        gsem,
        zbuf,
        qbuf,
        zsend,
        zrecv,
        ysend,
        yrecv,
        xsend,
        xrecv,
    ):
        my_x = lax.axis_index("x")
        my_y = lax.axis_index("y")
        my_z = lax.axis_index("z")
        y_peer = (my_x, 1 - my_y, my_z)
        x_peer = (1 - my_x, my_y, my_z)

        def gather_issue(h):
            def lp(t, c):
                idx = safe_ref[h * Th + t]
                pltpu.make_async_copy(
                    E_ref.at[idx], gbuf.at[h, t], gsem.at[h]
                ).start()
                return c

            lax.fori_loop(0, Th, lp, 0, unroll=8)

        def gather_finish(h):
            pltpu.make_async_copy(
                E_ref.at[pl.ds(0, Th)], gbuf.at[h], gsem.at[h]
            ).wait()
            zbuf[h, 0, :, :] = (
                gbuf[h, :, :] * mask_ref[h * Th : (h + 1) * Th, :]
            ).astype(jnp.bfloat16)

        barrier_sem = pltpu.get_barrier_semaphore()
        for dev in (
            (my_x, my_y, lax.rem(my_z + 1, N_Z)),
            (my_x, my_y, lax.rem(my_z + N_Z - 1, N_Z)),
            y_peer,
            x_peer,
        ):
            pl.semaphore_signal(
                barrier_sem, inc=1, device_id=dev, device_id_type=MESH
            )

        gather_issue(0)
        pl.semaphore_wait(barrier_sem, 4)

        def z_issue(h, s):
            src_r = 0 if s == 1 else s - 1
            src_l = 0 if s == 1 else N_Z - 1 + s - 1

            @pl.when((my_z < N_Z - 1) & (my_z >= s - 1))
            def _():
                pltpu.make_async_remote_copy(
                    src_ref=zbuf.at[h, src_r],
                    dst_ref=zbuf.at[h, s],
                    send_sem=zsend.at[h, s - 1, 0],
                    recv_sem=zrecv.at[h, s - 1, 0],
                    device_id=(my_x, my_y, my_z + 1),
                    device_id_type=MESH,
                ).start()

            @pl.when((my_z > 0) & (my_z <= N_Z - s))
            def _():
                pltpu.make_async_remote_copy(
                    src_ref=zbuf.at[h, src_l],
                    dst_ref=zbuf.at[h, N_Z - 1 + s],
                    send_sem=zsend.at[h, s - 1, 1],
                    recv_sem=zrecv.at[h, s - 1, 1],
                    device_id=(my_x, my_y, my_z - 1),
                    device_id_type=MESH,
                ).start()

        def z_wait(h, s):
            @pl.when(my_z >= s)
            def _():
                pltpu.make_async_remote_copy(
                    src_ref=zbuf.at[h, 0],
                    dst_ref=zbuf.at[h, s],
                    send_sem=zsend.at[h, s - 1, 0],
                    recv_sem=zrecv.at[h, s - 1, 0],
                    device_id=(my_x, my_y, my_z),
                    device_id_type=MESH,
                ).wait_recv()

            @pl.when(my_z <= N_Z - 1 - s)
            def _():
                pltpu.make_async_remote_copy(
                    src_ref=zbuf.at[h, 0],
                    dst_ref=zbuf.at[h, N_Z - 1 + s],
                    send_sem=zsend.at[h, s - 1, 1],
                    recv_sem=zrecv.at[h, s - 1, 1],
                    device_id=(my_x, my_y, my_z),
                    device_id_type=MESH,
                ).wait_recv()

            src_r = 0 if s == 1 else s - 1
            src_l = 0 if s == 1 else N_Z - 1 + s - 1

            @pl.when((my_z < N_Z - 1) & (my_z >= s - 1))
            def _():
                pltpu.make_async_remote_copy(
                    src_ref=zbuf.at[h, src_r],
                    dst_ref=zbuf.at[h, 0],
                    send_sem=zsend.at[h, s - 1, 0],
                    recv_sem=zrecv.at[h, s - 1, 0],
                    device_id=(my_x, my_y, my_z),
                    device_id_type=MESH,
                ).wait_send()

            @pl.when((my_z > 0) & (my_z <= N_Z - s))
            def _():
                pltpu.make_async_remote_copy(
                    src_ref=zbuf.at[h, src_l],
                    dst_ref=zbuf.at[h, 0],
                    send_sem=zsend.at[h, s - 1, 1],
                    recv_sem=zrecv.at[h, s - 1, 1],
                    device_id=(my_x, my_y, my_z),
                    device_id_type=MESH,
                ).wait_send()

        def qcopy(h, src_slot, dst_slot, sems_s, sems_r, si, peer):
            return pltpu.make_async_remote_copy(
                src_ref=qbuf.at[h, src_slot],
                dst_ref=qbuf.at[h, dst_slot],
                send_sem=sems_s.at[h, si],
                recv_sem=sems_r.at[h, si],
                device_id=peer,
                device_id_type=MESH,
            )

        quarters = (
            2 * my_x + my_y,
            2 * my_x + 1 - my_y,
            2 * (1 - my_x) + my_y,
            2 * (1 - my_x) + 1 - my_y,
        )

        def out_piece(h, k):
            out_ref[pl.ds(quarters[k] * Tq + h * Th, Th), :] = qbuf[
                h, k, :, :
            ].astype(jnp.float32)

        def half_sum(h):
            for zv in range(N_Z):

                @pl.when(my_z == zv)
                def _(zv=zv):
                    slots = (
                        [0]
                        + list(range(1, zv + 1))
                        + [N_Z - 1 + s for s in range(1, N_Z - zv)]
                    )
                    acc = zbuf[h, slots[0], :, :]
                    for sl in slots[1:]:
                        acc = acc + zbuf[h, sl, :, :]
                    qbuf[h, 0, :, :] = acc

            out_piece(h, 0)

        gather_finish(0)
        z_issue(0, 1)
        gather_issue(1)
        gather_finish(1)
        z_issue(1, 1)
        z_wait(0, 1)
        z_issue(0, 2)
        z_wait(1, 1)
        z_issue(1, 2)
        z_wait(0, 2)
        z_issue(0, 3)
        z_wait(1, 2)
        z_issue(1, 3)
        z_wait(0, 3)

        half_sum(0)
        qcopy(0, 0, 1, ysend, yrecv, 0, y_peer).start()
        qcopy(0, 0, 2, xsend, xrecv, 0, x_peer).start()

        z_wait(1, 3)

        half_sum(1)
        qcopy(1, 0, 1, ysend, yrecv, 0, y_peer).start()
        qcopy(1, 0, 2, xsend, xrecv, 0, x_peer).start()

        qcopy(0, 0, 1, ysend, yrecv, 0, y_peer).wait_recv()
        qcopy(0, 1, 3, xsend, xrecv, 1, x_peer).start()
        out_piece(0, 1)
        qcopy(1, 0, 2, xsend, xrecv, 0, x_peer).wait_recv()
        qcopy(1, 2, 3, ysend, yrecv, 1, y_peer).start()
        out_piece(1, 2)

        qcopy(0, 0, 2, xsend, xrecv, 0, x_peer).wait_recv()
        out_piece(0, 2)
        qcopy(1, 0, 1, ysend, yrecv, 0, y_peer).wait_recv()
        out_piece(1, 1)
        qcopy(0, 1, 3, xsend, xrecv, 1, x_peer).wait_recv()
        out_piece(0, 3)
        qcopy(1, 2, 3, ysend, yrecv, 1, y_peer).wait_recv()
        out_piece(1, 3)

        qcopy(0, 0, 0, ysend, yrecv, 0, y_peer).wait_send()
        qcopy(0, 0, 0, xsend, xrecv, 0, x_peer).wait_send()
        qcopy(0, 1, 0, xsend, xrecv, 1, x_peer).wait_send()
        qcopy(1, 0, 0, ysend, yrecv, 0, y_peer).wait_send()
        qcopy(1, 0, 0, xsend, xrecv, 0, x_peer).wait_send()
        qcopy(1, 2, 0, ysend, yrecv, 1, y_peer).wait_send()

    grid_spec = pltpu.PrefetchScalarGridSpec(
        num_scalar_prefetch=1,
        in_specs=[
            pl.BlockSpec(memory_space=pl.ANY),
            pl.BlockSpec(memory_space=pltpu.VMEM),
        ],
        out_specs=pl.BlockSpec(memory_space=pltpu.VMEM),
        scratch_shapes=[
            pltpu.VMEM((2, Th, D), jnp.float32),
            pltpu.SemaphoreType.DMA((2,)),
            pltpu.VMEM((2, NSLOT, Th, D), jnp.bfloat16),
            pltpu.VMEM((2, 4, Th, D), jnp.bfloat16),
            pltpu.SemaphoreType.DMA((2, N_Z - 1, 2)),
            pltpu.SemaphoreType.DMA((2, N_Z - 1, 2)),
            pltpu.SemaphoreType.DMA((2, 2)),
            pltpu.SemaphoreType.DMA((2, 2)),
            pltpu.SemaphoreType.DMA((2, 2)),
            pltpu.SemaphoreType.DMA((2, 2)),
        ],
    )

    return pl.pallas_call(
        body,
        out_shape=jax.ShapeDtypeStruct((T, D), jnp.float32),
        grid_spec=grid_spec,
        compiler_params=pltpu.CompilerParams(collective_id=0),
    )(safe, E, maskf)


# device time: 60219 ns/iter; 3.9057x vs baseline; 1.1624x over previous
import jax
import jax.numpy as jnp
from jax import lax
from jax.experimental import pallas as pl
from jax.experimental.pallas import tpu as pltpu

N_Z = 4
CH = 4
NSLOT = 2 * N_Z - 1
MESH = pl.DeviceIdType.MESH


def kernel(ids, E):
    T = ids.shape[0]
    V_loc, D = E.shape
    Tq = T // 4
    Tc = Tq // CH

    x = lax.axis_index("x")
    y = lax.axis_index("y")
    z = lax.axis_index("z")
    q = 2 * x + y

    ids_q = lax.dynamic_slice(ids, (q * Tq,), (Tq,))
    local = ids_q - z * V_loc
    mask = (local >= 0) & (local < V_loc)
    safe = jnp.where(mask, local, 0).astype(jnp.int32)
    maskf = mask.astype(jnp.float32)[:, None]

    def body(
        safe_ref,
        E_ref,
        mask_ref,
        out_ref,
        gbuf,
        gsem,
        zbuf,
        qbuf,
        zsend,
        zrecv,
        ysend,
        yrecv,
        xsend,
        xrecv,
    ):
        my_x = lax.axis_index("x")
        my_y = lax.axis_index("y")
        my_z = lax.axis_index("z")
        y_peer = (my_x, 1 - my_y, my_z)
        x_peer = (1 - my_x, my_y, my_z)

        def gather_issue(c):
            def lp(t, acc):
                idx = safe_ref[c * Tc + t]
                pltpu.make_async_copy(
                    E_ref.at[idx], gbuf.at[c, t], gsem.at[c]
                ).start()
                return acc

            lax.fori_loop(0, Tc, lp, 0, unroll=8)

        def gather_finish(c):
            pltpu.make_async_copy(
                E_ref.at[pl.ds(0, Tc)], gbuf.at[c], gsem.at[c]
            ).wait()
            zbuf[c, 0, :, :] = (
                gbuf[c, :, :] * mask_ref[c * Tc : (c + 1) * Tc, :]
            ).astype(jnp.bfloat16)

        barrier_sem = pltpu.get_barrier_semaphore()
        for dev in (
            (my_x, my_y, lax.rem(my_z + 1, N_Z)),
            (my_x, my_y, lax.rem(my_z + N_Z - 1, N_Z)),
            y_peer,
            x_peer,
        ):
            pl.semaphore_signal(
                barrier_sem, inc=1, device_id=dev, device_id_type=MESH
            )

        def z_issue(c, s):
            src_r = 0 if s == 1 else s - 1
            src_l = 0 if s == 1 else N_Z - 1 + s - 1

            @pl.when((my_z < N_Z - 1) & (my_z >= s - 1))
            def _():
                pltpu.make_async_remote_copy(
                    src_ref=zbuf.at[c, src_r],
                    dst_ref=zbuf.at[c, s],
                    send_sem=zsend.at[c, s - 1, 0],
                    recv_sem=zrecv.at[c, s - 1, 0],
                    device_id=(my_x, my_y, my_z + 1),
                    device_id_type=MESH,
                ).start()

            @pl.when((my_z > 0) & (my_z <= N_Z - s))
            def _():
                pltpu.make_async_remote_copy(
                    src_ref=zbuf.at[c, src_l],
                    dst_ref=zbuf.at[c, N_Z - 1 + s],
                    send_sem=zsend.at[c, s - 1, 1],
                    recv_sem=zrecv.at[c, s - 1, 1],
                    device_id=(my_x, my_y, my_z - 1),
                    device_id_type=MESH,
                ).start()

        def z_wait(c, s):
            @pl.when(my_z >= s)
            def _():
                pltpu.make_async_remote_copy(
                    src_ref=zbuf.at[c, 0],
                    dst_ref=zbuf.at[c, s],
                    send_sem=zsend.at[c, s - 1, 0],
                    recv_sem=zrecv.at[c, s - 1, 0],
                    device_id=(my_x, my_y, my_z),
                    device_id_type=MESH,
                ).wait_recv()

            @pl.when(my_z <= N_Z - 1 - s)
            def _():
                pltpu.make_async_remote_copy(
                    src_ref=zbuf.at[c, 0],
                    dst_ref=zbuf.at[c, N_Z - 1 + s],
                    send_sem=zsend.at[c, s - 1, 1],
                    recv_sem=zrecv.at[c, s - 1, 1],
                    device_id=(my_x, my_y, my_z),
                    device_id_type=MESH,
                ).wait_recv()

            src_r = 0 if s == 1 else s - 1
            src_l = 0 if s == 1 else N_Z - 1 + s - 1

            @pl.when((my_z < N_Z - 1) & (my_z >= s - 1))
            def _():
                pltpu.make_async_remote_copy(
                    src_ref=zbuf.at[c, src_r],
                    dst_ref=zbuf.at[c, 0],
                    send_sem=zsend.at[c, s - 1, 0],
                    recv_sem=zrecv.at[c, s - 1, 0],
                    device_id=(my_x, my_y, my_z),
                    device_id_type=MESH,
                ).wait_send()

            @pl.when((my_z > 0) & (my_z <= N_Z - s))
            def _():
                pltpu.make_async_remote_copy(
                    src_ref=zbuf.at[c, src_l],
                    dst_ref=zbuf.at[c, 0],
                    send_sem=zsend.at[c, s - 1, 1],
                    recv_sem=zrecv.at[c, s - 1, 1],
                    device_id=(my_x, my_y, my_z),
                    device_id_type=MESH,
                ).wait_send()

        def qcopy(c, src_slot, dst_slot, sems_s, sems_r, si, peer):
            return pltpu.make_async_remote_copy(
                src_ref=qbuf.at[c, src_slot],
                dst_ref=qbuf.at[c, dst_slot],
                send_sem=sems_s.at[c, si],
                recv_sem=sems_r.at[c, si],
                device_id=peer,
                device_id_type=MESH,
            )

        quarters = (
            2 * my_x + my_y,
            2 * my_x + 1 - my_y,
            2 * (1 - my_x) + my_y,
            2 * (1 - my_x) + 1 - my_y,
        )

        def out_piece(c, k):
            out_ref[pl.ds(quarters[k] * Tq + c * Tc, Tc), :] = qbuf[
                c, k, :, :
            ].astype(jnp.float32)

        def tail_start(c):
            for zv in range(N_Z):

                @pl.when(my_z == zv)
                def _(zv=zv):
                    slots = (
                        [0]
                        + list(range(1, zv + 1))
                        + [N_Z - 1 + s for s in range(1, N_Z - zv)]
                    )
                    acc = zbuf[c, slots[0], :, :]
                    for sl in slots[1:]:
                        acc = acc + zbuf[c, sl, :, :]
                    qbuf[c, 0, :, :] = acc

            out_piece(c, 0)
            qcopy(c, 0, 1, ysend, yrecv, 0, y_peer).start()
            qcopy(c, 0, 2, xsend, xrecv, 0, x_peer).start()

        def diag_forward(c):
            if c % 2 == 0:
                qcopy(c, 0, 1, ysend, yrecv, 0, y_peer).wait_recv()
                qcopy(c, 1, 3, xsend, xrecv, 1, x_peer).start()
                out_piece(c, 1)
            else:
                qcopy(c, 0, 2, xsend, xrecv, 0, x_peer).wait_recv()
                qcopy(c, 2, 3, ysend, yrecv, 1, y_peer).start()
                out_piece(c, 2)

        def tail_finish(c):
            if c % 2 == 0:
                qcopy(c, 0, 2, xsend, xrecv, 0, x_peer).wait_recv()
                out_piece(c, 2)
                qcopy(c, 1, 3, xsend, xrecv, 1, x_peer).wait_recv()
                out_piece(c, 3)
                qcopy(c, 0, 0, ysend, yrecv, 0, y_peer).wait_send()
                qcopy(c, 0, 0, xsend, xrecv, 0, x_peer).wait_send()
                qcopy(c, 1, 0, xsend, xrecv, 1, x_peer).wait_send()
            else:
                qcopy(c, 0, 1, ysend, yrecv, 0, y_peer).wait_recv()
                out_piece(c, 1)
                qcopy(c, 2, 3, ysend, yrecv, 1, y_peer).wait_recv()
                out_piece(c, 3)
                qcopy(c, 0, 0, ysend, yrecv, 0, y_peer).wait_send()
                qcopy(c, 0, 0, xsend, xrecv, 0, x_peer).wait_send()
                qcopy(c, 2, 0, ysend, yrecv, 1, y_peer).wait_send()

        gather_issue(0)
        pl.semaphore_wait(barrier_sem, 4)
        gather_finish(0)
        z_issue(0, 1)
        gather_issue(1)
        gather_finish(1)
        z_issue(1, 1)
        z_wait(0, 1)
        z_issue(0, 2)
        gather_issue(2)
        gather_finish(2)
        z_issue(2, 1)
        z_wait(1, 1)
        z_issue(1, 2)
        gather_issue(3)
        gather_finish(3)
        z_issue(3, 1)
        z_wait(0, 2)
        z_issue(0, 3)
        z_wait(2, 1)
        z_issue(2, 2)
        z_wait(1, 2)
        z_issue(1, 3)
        z_wait(3, 1)
        z_issue(3, 2)
        z_wait(0, 3)
        tail_start(0)
        z_wait(2, 2)
        z_issue(2, 3)
        z_wait(1, 3)
        tail_start(1)
        z_wait(3, 2)
        z_issue(3, 3)
        diag_forward(0)
        z_wait(2, 3)
        tail_start(2)
        diag_forward(1)
        z_wait(3, 3)
        tail_start(3)
        diag_forward(2)
        diag_forward(3)
        tail_finish(0)
        tail_finish(1)
        tail_finish(2)
        tail_finish(3)

    grid_spec = pltpu.PrefetchScalarGridSpec(
        num_scalar_prefetch=1,
        in_specs=[
            pl.BlockSpec(memory_space=pl.ANY),
            pl.BlockSpec(memory_space=pltpu.VMEM),
        ],
        out_specs=pl.BlockSpec(memory_space=pltpu.VMEM),
        scratch_shapes=[
            pltpu.VMEM((CH, Tc, D), jnp.float32),
            pltpu.SemaphoreType.DMA((CH,)),
            pltpu.VMEM((CH, NSLOT, Tc, D), jnp.bfloat16),
            pltpu.VMEM((CH, 4, Tc, D), jnp.bfloat16),
            pltpu.SemaphoreType.DMA((CH, N_Z - 1, 2)),
            pltpu.SemaphoreType.DMA((CH, N_Z - 1, 2)),
            pltpu.SemaphoreType.DMA((CH, 2)),
            pltpu.SemaphoreType.DMA((CH, 2)),
            pltpu.SemaphoreType.DMA((CH, 2)),
            pltpu.SemaphoreType.DMA((CH, 2)),
        ],
    )

    return pl.pallas_call(
        body,
        out_shape=jax.ShapeDtypeStruct((T, D), jnp.float32),
        grid_spec=grid_spec,
        compiler_params=pltpu.CompilerParams(collective_id=0),
    )(safe, E, maskf)


# device time: 58636 ns/iter; 4.0111x vs baseline; 1.0270x over previous
import jax
import jax.numpy as jnp
from jax import lax
from jax.experimental import pallas as pl
from jax.experimental.pallas import tpu as pltpu

N_Z = 4
CH = 4
NSLOT = 5
MESH = pl.DeviceIdType.MESH


def kernel(ids, E):
    T = ids.shape[0]
    V_loc, D = E.shape
    Tq = T // 4
    Tc = Tq // CH

    x = lax.axis_index("x")
    y = lax.axis_index("y")
    z = lax.axis_index("z")
    q = 2 * x + y

    ids_q = lax.dynamic_slice(ids, (q * Tq,), (Tq,))
    local = ids_q - z * V_loc
    mask = (local >= 0) & (local < V_loc)
    safe = jnp.where(mask, local, 0).astype(jnp.int32)
    maskf = mask.astype(jnp.float32)[:, None]

    def body(
        safe_ref,
        E_ref,
        mask_ref,
        out_ref,
        gbuf,
        gsem,
        zbuf,
        qbuf,
        zsend,
        zrecv,
        ysend,
        yrecv,
        xsend,
        xrecv,
    ):
        my_x = lax.axis_index("x")
        my_y = lax.axis_index("y")
        my_z = lax.axis_index("z")
        y_peer = (my_x, 1 - my_y, my_z)
        x_peer = (1 - my_x, my_y, my_z)

        def gather_issue(c):
            def lp(t, acc):
                idx = safe_ref[c * Tc + t]
                pltpu.make_async_copy(
                    E_ref.at[idx], gbuf.at[c, t], gsem.at[c]
                ).start()
                return acc

            lax.fori_loop(0, Tc, lp, 0, unroll=8)

        def gather_finish(c):
            pltpu.make_async_copy(
                E_ref.at[pl.ds(0, Tc)], gbuf.at[c], gsem.at[c]
            ).wait()
            zbuf[c, 0, :, :] = (
                gbuf[c, :, :] * mask_ref[c * Tc : (c + 1) * Tc, :]
            ).astype(jnp.bfloat16)

        barrier_sem = pltpu.get_barrier_semaphore()
        for dev in (
            (my_x, my_y, lax.rem(my_z + 1, N_Z)),
            (my_x, my_y, lax.rem(my_z + N_Z - 1, N_Z)),
            y_peer,
            x_peer,
        ):
            pl.semaphore_signal(
                barrier_sem, inc=1, device_id=dev, device_id_type=MESH
            )

        def zcopy(c, src_slot, dst_slot, dirn, dev):
            return pltpu.make_async_remote_copy(
                src_ref=zbuf.at[c, src_slot],
                dst_ref=zbuf.at[c, dst_slot],
                send_sem=zsend.at[c, dirn],
                recv_sem=zrecv.at[c, dirn],
                device_id=dev,
                device_id_type=MESH,
            )

        def z_edge(c):
            @pl.when(my_z == 0)
            def _():
                zcopy(c, 0, 1, 0, (my_x, my_y, my_z + 1)).start()

            @pl.when(my_z == N_Z - 1)
            def _():
                zcopy(c, 0, 2, 1, (my_x, my_y, my_z - 1)).start()

        def z_mid(c):
            def rfwd():
                zcopy(c, 0, 1, 0, (my_x, my_y, my_z)).wait_recv()
                zbuf[c, 3, :, :] = zbuf[c, 0, :, :] + zbuf[c, 1, :, :]
                zcopy(c, 3, 1, 0, (my_x, my_y, my_z + 1)).start()

            def lfwd():
                zcopy(c, 0, 2, 1, (my_x, my_y, my_z)).wait_recv()
                zbuf[c, 4, :, :] = zbuf[c, 0, :, :] + zbuf[c, 2, :, :]
                zcopy(c, 4, 2, 1, (my_x, my_y, my_z - 1)).start()

            @pl.when(my_z == 1)
            def _():
                rfwd()
                lfwd()

            @pl.when(my_z == 2)
            def _():
                lfwd()
                rfwd()

        def qcopy(c, src_slot, dst_slot, sems_s, sems_r, si, peer):
            return pltpu.make_async_remote_copy(
                src_ref=qbuf.at[c, src_slot],
                dst_ref=qbuf.at[c, dst_slot],
                send_sem=sems_s.at[c, si],
                recv_sem=sems_r.at[c, si],
                device_id=peer,
                device_id_type=MESH,
            )

        quarters = (
            2 * my_x + my_y,
            2 * my_x + 1 - my_y,
            2 * (1 - my_x) + my_y,
            2 * (1 - my_x) + 1 - my_y,
        )

        def out_piece(c, k):
            out_ref[pl.ds(quarters[k] * Tq + c * Tc, Tc), :] = qbuf[
                c, k, :, :
            ].astype(jnp.float32)

        def tail_start(c):
            @pl.when(my_z == 0)
            def _():
                zcopy(c, 0, 2, 1, (my_x, my_y, my_z)).wait_recv()
                qbuf[c, 0, :, :] = zbuf[c, 0, :, :] + zbuf[c, 2, :, :]

            @pl.when(my_z == N_Z - 1)
            def _():
                zcopy(c, 0, 1, 0, (my_x, my_y, my_z)).wait_recv()
                qbuf[c, 0, :, :] = zbuf[c, 0, :, :] + zbuf[c, 1, :, :]

            @pl.when((my_z > 0) & (my_z < N_Z - 1))
            def _():
                qbuf[c, 0, :, :] = (
                    zbuf[c, 0, :, :] + zbuf[c, 1, :, :] + zbuf[c, 2, :, :]
                )

            out_piece(c, 0)
            qcopy(c, 0, 1, ysend, yrecv, 0, y_peer).start()
            qcopy(c, 0, 2, xsend, xrecv, 0, x_peer).start()

            @pl.when(my_z == 0)
            def _():
                zcopy(c, 0, 1, 0, (my_x, my_y, my_z)).wait_send()

            @pl.when((my_z > 0) & (my_z < N_Z - 1))
            def _():
                zcopy(c, 3, 1, 0, (my_x, my_y, my_z)).wait_send()
                zcopy(c, 4, 2, 1, (my_x, my_y, my_z)).wait_send()

            @pl.when(my_z == N_Z - 1)
            def _():
                zcopy(c, 0, 2, 1, (my_x, my_y, my_z)).wait_send()

        def diag_forward(c):
            if c % 2 == 0:
                qcopy(c, 0, 1, ysend, yrecv, 0, y_peer).wait_recv()
                qcopy(c, 1, 3, xsend, xrecv, 1, x_peer).start()
                out_piece(c, 1)
            else:
                qcopy(c, 0, 2, xsend, xrecv, 0, x_peer).wait_recv()
                qcopy(c, 2, 3, ysend, yrecv, 1, y_peer).start()
                out_piece(c, 2)

        def tail_finish(c):
            if c % 2 == 0:
                qcopy(c, 0, 2, xsend, xrecv, 0, x_peer).wait_recv()
                out_piece(c, 2)
                qcopy(c, 1, 3, xsend, xrecv, 1, x_peer).wait_recv()
                out_piece(c, 3)
                qcopy(c, 0, 0, ysend, yrecv, 0, y_peer).wait_send()
                qcopy(c, 0, 0, xsend, xrecv, 0, x_peer).wait_send()
                qcopy(c, 1, 0, xsend, xrecv, 1, x_peer).wait_send()
            else:
                qcopy(c, 0, 1, ysend, yrecv, 0, y_peer).wait_recv()
                out_piece(c, 1)
                qcopy(c, 2, 3, ysend, yrecv, 1, y_peer).wait_recv()
                out_piece(c, 3)
                qcopy(c, 0, 0, ysend, yrecv, 0, y_peer).wait_send()
                qcopy(c, 0, 0, xsend, xrecv, 0, x_peer).wait_send()
                qcopy(c, 2, 0, ysend, yrecv, 1, y_peer).wait_send()

        gather_issue(0)
        pl.semaphore_wait(barrier_sem, 4)
        gather_finish(0)
        z_edge(0)
        gather_issue(1)
        gather_finish(1)
        z_edge(1)
        z_mid(0)
        gather_issue(2)
        gather_finish(2)
        z_edge(2)
        z_mid(1)
        gather_issue(3)
        gather_finish(3)
        z_edge(3)
        z_mid(2)
        tail_start(0)
        z_mid(3)
        tail_start(1)
        diag_forward(0)
        tail_start(2)
        diag_forward(1)
        tail_start(3)
        diag_forward(2)
        diag_forward(3)
        tail_finish(0)
        tail_finish(1)
        tail_finish(2)
        tail_finish(3)

    grid_spec = pltpu.PrefetchScalarGridSpec(
        num_scalar_prefetch=1,
        in_specs=[
            pl.BlockSpec(memory_space=pl.ANY),
            pl.BlockSpec(memory_space=pltpu.VMEM),
        ],
        out_specs=pl.BlockSpec(memory_space=pltpu.VMEM),
        scratch_shapes=[
            pltpu.VMEM((CH, Tc, D), jnp.float32),
            pltpu.SemaphoreType.DMA((CH,)),
            pltpu.VMEM((CH, NSLOT, Tc, D), jnp.bfloat16),
            pltpu.VMEM((CH, 4, Tc, D), jnp.bfloat16),
            pltpu.SemaphoreType.DMA((CH, 2)),
            pltpu.SemaphoreType.DMA((CH, 2)),
            pltpu.SemaphoreType.DMA((CH, 2)),
            pltpu.SemaphoreType.DMA((CH, 2)),
            pltpu.SemaphoreType.DMA((CH, 2)),
            pltpu.SemaphoreType.DMA((CH, 2)),
        ],
    )

    return pl.pallas_call(
        body,
        out_shape=jax.ShapeDtypeStruct((T, D), jnp.float32),
        grid_spec=grid_spec,
        compiler_params=pltpu.CompilerParams(collective_id=0),
    )(safe, E, maskf)
